# baseline (speedup 1.0000x reference)
import functools

import jax
import jax.numpy as jnp
from jax import lax
from jax.experimental import pallas as pl
from jax.experimental.pallas import tpu as pltpu
from jax.experimental.pallas import tpu_sc as plsc

B = 4
T = 8192
D = 2048
NUM_EXPERTS = 64
TOP_K = 8

TC_T = 5120
SC_T = T - TC_T
SUB_PER_B = 8
ROWS_PER_SUB = SC_T // SUB_PER_B
CH = 16
N_CHUNKS = ROWS_PER_SUB // CH
T_BLK = 128


def _tree_sum(vals):
    while len(vals) > 1:
        nxt = [vals[2 * i] + vals[2 * i + 1] for i in range(len(vals) // 2)]
        if len(vals) % 2:
            nxt.append(vals[-1])
        vals = nxt
    return vals[0]


def _sc_reduce_body(x_hbm, out_hbm, buf, acc, sem):
    c = lax.axis_index("c")
    s = lax.axis_index("s")
    wid = s * 2 + c
    batch = wid // SUB_PER_B
    sub = wid % SUB_PER_B
    row0 = batch * T + TC_T + sub * ROWS_PER_SUB

    def dma(chunk_idx, slot):
        return pltpu.make_async_copy(
            x_hbm.at[pl.ds(row0 + chunk_idx * CH, CH), :],
            buf.at[slot],
            sem.at[slot],
        )

    dma(0, 0).start()

    def zbody(j, carry):
        acc[pl.ds(j * 16, 16)] = jnp.zeros((16,), jnp.float32)
        return carry

    lax.fori_loop(0, D // 16, zbody, 0)

    def chunk_pair(g, carry):
        for b_ in range(2):
            t_ = g * 2 + b_

            @pl.when(t_ + 1 < N_CHUNKS)
            def _start_next():
                dma(t_ + 1, (b_ + 1) % 2).start()

            dma(t_, b_).wait()

            def jbody(j, jcarry):
                base = j * 16
                vals = [buf[b_, i, pl.ds(base, 16)] for i in range(CH)]
                acc[pl.ds(base, 16)] += _tree_sum(vals)
                return jcarry

            lax.fori_loop(0, D // 16, jbody, 0)
        return carry

    lax.fori_loop(0, N_CHUNKS // 2, chunk_pair, 0)
    pltpu.sync_copy(acc, out_hbm.at[batch, sub])


def _sc_reduce(x_flat):
    mesh = plsc.VectorSubcoreMesh(core_axis_name="c", subcore_axis_name="s")
    return pl.kernel(
        _sc_reduce_body,
        mesh=mesh,
        out_type=jax.ShapeDtypeStruct((B, SUB_PER_B, D), jnp.float32),
        scratch_types=[
            pltpu.VMEM((2, CH, D), jnp.float32),
            pltpu.VMEM((D,), jnp.float32),
            pltpu.SemaphoreType.DMA((2,)),
        ],
    )(x_flat)


def _tc_head_kernel(x_ref, out_ref, acc_ref, *, n_blocks):
    t = pl.program_id(0)

    @pl.when(t == 0)
    def _init():
        acc_ref[...] = jnp.zeros_like(acc_ref)

    acc_ref[...] += jnp.sum(x_ref[...], axis=1)

    @pl.when(t == n_blocks - 1)
    def _fin():
        out_ref[...] = acc_ref[...]


def _combine_kernel(tc_ref, sc_ref, w_ref, b_ref, sw_ref, idx_ref):
    pooled = (tc_ref[...] + jnp.sum(sc_ref[...], axis=1)) * (1.0 / T)
    logits = jax.lax.dot_general(
        pooled.astype(jnp.bfloat16), w_ref[...].astype(jnp.bfloat16),
        (((1,), (1,)), ((), ())),
        preferred_element_type=jnp.float32,
    ) + b_ref[...]

    e_iota = jax.lax.broadcasted_iota(jnp.int32, logits.shape, 1)
    vals = logits
    top_vals = []
    top_idx = []
    for _ in range(TOP_K):
        m = jnp.max(vals, axis=1, keepdims=True)
        i = jnp.min(jnp.where(vals == m, e_iota, NUM_EXPERTS),
                    axis=1, keepdims=True)
        top_vals.append(m)
        top_idx.append(i)
        vals = jnp.where(e_iota == i, -jnp.inf, vals)

    tv = jnp.concatenate(top_vals, axis=1)
    ex = jnp.exp(tv - tv[:, :1])
    probs = ex / jnp.sum(ex, axis=1, keepdims=True)

    sparse = jnp.zeros_like(logits)
    for k in range(TOP_K):
        sparse += jnp.where(e_iota == top_idx[k], probs[:, k:k + 1], 0.0)

    sw_ref[...] = sparse
    idx_ref[...] = jnp.concatenate(top_idx, axis=1)


@jax.jit
def kernel(x, W, b):
    x_flat = x.reshape(B * T, D)
    sc_part = _sc_reduce(x_flat)

    n_blocks = TC_T // T_BLK
    tc_part = pl.pallas_call(
        functools.partial(_tc_head_kernel, n_blocks=n_blocks),
        grid=(n_blocks,),
        in_specs=[pl.BlockSpec((B, T_BLK, D), lambda t: (0, t, 0))],
        out_specs=pl.BlockSpec((B, D), lambda t: (0, 0)),
        out_shape=jax.ShapeDtypeStruct((B, D), jnp.float32),
        scratch_shapes=[pltpu.VMEM((B, D), jnp.float32)],
    )(x)

    sw, idx = pl.pallas_call(
        _combine_kernel,
        in_specs=[
            pl.BlockSpec((B, D), lambda: (0, 0)),
            pl.BlockSpec((B, SUB_PER_B, D), lambda: (0, 0, 0)),
            pl.BlockSpec((NUM_EXPERTS, D), lambda: (0, 0)),
            pl.BlockSpec((1, NUM_EXPERTS), lambda: (0, 0)),
        ],
        out_specs=[
            pl.BlockSpec((B, NUM_EXPERTS), lambda: (0, 0)),
            pl.BlockSpec((B, TOP_K), lambda: (0, 0)),
        ],
        out_shape=[
            jax.ShapeDtypeStruct((B, NUM_EXPERTS), jnp.float32),
            jax.ShapeDtypeStruct((B, TOP_K), jnp.int32),
        ],
    )(tc_part, sc_part, W, b.reshape(1, NUM_EXPERTS))
    return (sw, idx)


# speedup vs baseline: 1.1590x; 1.1590x over previous
import jax
import jax.numpy as jnp
from jax.experimental import pallas as pl
from jax.experimental.pallas import tpu as pltpu

NUM_EXPERTS = 64
TOP_K = 8
T_BLK = 128
NBUF = 8


def _gating_kernel(x_hbm, w_ref, b_ref, sw_ref, idx_ref, buf, acc_ref, sem):
    B, T, D = x_hbm.shape
    n_steps = T // T_BLK

    def copy(step, slot):
        return pltpu.make_async_copy(
            x_hbm.at[:, pl.ds(step * T_BLK, T_BLK), :],
            buf.at[slot],
            sem.at[slot],
        )

    for s in range(min(NBUF, n_steps)):
        copy(s, s).start()

    acc_ref[...] = jnp.zeros_like(acc_ref)
    for step in range(n_steps):
        slot = step % NBUF
        copy(step, slot).wait()
        acc_ref[...] += jnp.sum(buf[slot], axis=1)
        nxt = step + NBUF
        if nxt < n_steps:
            copy(nxt, slot).start()

    pooled = acc_ref[...] * (1.0 / T)
    logits = jax.lax.dot_general(
        pooled.astype(jnp.bfloat16), w_ref[...].astype(jnp.bfloat16),
        (((1,), (1,)), ((), ())),
        preferred_element_type=jnp.float32,
    ) + b_ref[...]

    e_iota = jax.lax.broadcasted_iota(jnp.int32, logits.shape, 1)
    vals = logits
    top_vals = []
    top_idx = []
    for _ in range(TOP_K):
        m = jnp.max(vals, axis=1, keepdims=True)
        i = jnp.min(jnp.where(vals == m, e_iota, NUM_EXPERTS),
                    axis=1, keepdims=True)
        top_vals.append(m)
        top_idx.append(i)
        vals = jnp.where(e_iota == i, -jnp.inf, vals)

    tv = jnp.concatenate(top_vals, axis=1)
    ex = jnp.exp(tv - tv[:, :1])
    probs = ex / jnp.sum(ex, axis=1, keepdims=True)

    sparse = jnp.zeros_like(logits)
    for k in range(TOP_K):
        sparse += jnp.where(e_iota == top_idx[k], probs[:, k:k + 1], 0.0)

    sw_ref[...] = sparse
    idx_ref[...] = jnp.concatenate(top_idx, axis=1)


@jax.jit
def kernel(x, W, b):
    B, T, D = x.shape
    sw, idx = pl.pallas_call(
        _gating_kernel,
        in_specs=[
            pl.BlockSpec(memory_space=pl.ANY),
            pl.BlockSpec((NUM_EXPERTS, D), lambda: (0, 0)),
            pl.BlockSpec((1, NUM_EXPERTS), lambda: (0, 0)),
        ],
        out_specs=[
            pl.BlockSpec((B, NUM_EXPERTS), lambda: (0, 0)),
            pl.BlockSpec((B, TOP_K), lambda: (0, 0)),
        ],
        out_shape=[
            jax.ShapeDtypeStruct((B, NUM_EXPERTS), jnp.float32),
            jax.ShapeDtypeStruct((B, TOP_K), jnp.int32),
        ],
        scratch_shapes=[
            pltpu.VMEM((NBUF, B, T_BLK, D), jnp.float32),
            pltpu.VMEM((B, D), jnp.float32),
            pltpu.SemaphoreType.DMA((NBUF,)),
        ],
    )(x, W, b.reshape(1, NUM_EXPERTS))
    return (sw, idx)


# speedup vs baseline: 1.1941x; 1.0303x over previous
import functools

import jax
import jax.numpy as jnp
from jax.experimental import pallas as pl
from jax.experimental.pallas import tpu as pltpu

NUM_EXPERTS = 64
TOP_K = 8
T_BLK = 512


def _gating_kernel(x_ref, w_ref, b_ref, sw_ref, idx_ref, acc_ref, *, nt, nb, seq_len):
    t = pl.program_id(0)
    b = pl.program_id(1)

    @pl.when((t == 0) & (b == 0))
    def _init():
        acc_ref[...] = jnp.zeros_like(acc_ref)

    acc_ref[pl.ds(b, 1), :] += jnp.sum(x_ref[...], axis=1)

    @pl.when((t == nt - 1) & (b == nb - 1))
    def _finish():
        pooled = acc_ref[...] * (1.0 / seq_len)
        logits = jax.lax.dot_general(
            pooled.astype(jnp.bfloat16), w_ref[...].astype(jnp.bfloat16),
            (((1,), (1,)), ((), ())),
            preferred_element_type=jnp.float32,
        ) + b_ref[...]

        e_iota = jax.lax.broadcasted_iota(jnp.int32, logits.shape, 1)
        vals = logits
        top_vals = []
        top_idx = []
        for _ in range(TOP_K):
            m = jnp.max(vals, axis=1, keepdims=True)
            i = jnp.min(jnp.where(vals == m, e_iota, NUM_EXPERTS),
                        axis=1, keepdims=True)
            top_vals.append(m)
            top_idx.append(i)
            vals = jnp.where(e_iota == i, -jnp.inf, vals)

        tv = jnp.concatenate(top_vals, axis=1)
        ex = jnp.exp(tv - tv[:, :1])
        probs = ex / jnp.sum(ex, axis=1, keepdims=True)

        sparse = jnp.zeros_like(logits)
        for k in range(TOP_K):
            sparse += jnp.where(e_iota == top_idx[k], probs[:, k:k + 1], 0.0)

        sw_ref[...] = sparse
        idx_ref[...] = jnp.concatenate(top_idx, axis=1)


@jax.jit
def kernel(x, W, b):
    B, T, D = x.shape
    nt = T // T_BLK
    sw, idx = pl.pallas_call(
        functools.partial(_gating_kernel, nt=nt, nb=B, seq_len=T),
        grid=(nt, B),
        in_specs=[
            pl.BlockSpec((1, T_BLK, D), lambda t, b: (b, t, 0)),
            pl.BlockSpec((NUM_EXPERTS, D), lambda t, b: (0, 0)),
            pl.BlockSpec((1, NUM_EXPERTS), lambda t, b: (0, 0)),
        ],
        out_specs=[
            pl.BlockSpec((B, NUM_EXPERTS), lambda t, b: (0, 0)),
            pl.BlockSpec((B, TOP_K), lambda t, b: (0, 0)),
        ],
        out_shape=[
            jax.ShapeDtypeStruct((B, NUM_EXPERTS), jnp.float32),
            jax.ShapeDtypeStruct((B, TOP_K), jnp.int32),
        ],
        scratch_shapes=[pltpu.VMEM((B, D), jnp.float32)],
    )(x, W, b.reshape(1, NUM_EXPERTS))
    return (sw, idx)
